# baseline (speedup 1.0000x reference)
import functools

import jax
import jax.numpy as jnp
import numpy as np
from jax import lax
from jax.experimental import pallas as pl
from jax.experimental.pallas import tpu as pltpu
from jax.experimental.pallas import tpu_sc as plsc


def _make_sc_gather(V, D, B, L):
  info = plsc.get_sparse_core_info()
  NC, NS = info.num_cores, info.num_subcores
  NW = NC * NS
  n_seq = B * L
  assert n_seq % NW == 0 and B % NW == 0
  seq_per_w = n_seq // NW
  C = 1024
  assert seq_per_w % (2 * C) == 0
  n_pairs = seq_per_w // (2 * C)
  b_per_w = B // NW

  mesh = plsc.VectorSubcoreMesh(core_axis_name="c", subcore_axis_name="s")

  @functools.partial(
      pl.kernel, mesh=mesh,
      out_type=(
          jax.ShapeDtypeStruct((B, D), jnp.float32),
          jax.ShapeDtypeStruct((B, D), jnp.float32),
          jax.ShapeDtypeStruct((n_seq, D), jnp.float32),
      ),
      scratch_types=[
          pltpu.VMEM((C,), jnp.int32),
          pltpu.VMEM((C,), jnp.int32),
          pltpu.VMEM((C, D), jnp.float32),
          pltpu.VMEM((C, D), jnp.float32),
          pltpu.VMEM((b_per_w,), jnp.int32),
          pltpu.VMEM((b_per_w, D), jnp.float32),
          pltpu.SemaphoreType.DMA,
          pltpu.SemaphoreType.DMA,
      ],
      compiler_params=pltpu.CompilerParams(use_tc_tiling_on_sc=False),
  )
  def gather_k(table, uid, tid, sidx, e_user, e_tgt, seq_out,
               idx_a, idx_b, rows_a, rows_b, sid_v, srow_v, sem_a, sem_b):
    wid = lax.axis_index("s") * NC + lax.axis_index("c")
    ub = wid * b_per_w
    pltpu.sync_copy(uid.at[pl.ds(ub, b_per_w)], sid_v)
    pltpu.async_copy(table.at[sid_v], srow_v, sem_a).wait()
    pltpu.sync_copy(srow_v, e_user.at[pl.ds(ub, b_per_w)])
    pltpu.sync_copy(tid.at[pl.ds(ub, b_per_w)], sid_v)
    pltpu.async_copy(table.at[sid_v], srow_v, sem_a).wait()
    pltpu.sync_copy(srow_v, e_tgt.at[pl.ds(ub, b_per_w)])

    sb = wid * seq_per_w
    last = sb + seq_per_w - C

    pltpu.sync_copy(sidx.at[pl.ds(sb, C)], idx_a)
    ga0 = pltpu.async_copy(table.at[idx_a], rows_a, sem_a)

    def body(j, carry):
      c0 = sb + (2 * j) * C
      c1 = c0 + C
      c2 = jnp.minimum(c0 + 2 * C, last)
      pltpu.sync_copy(sidx.at[pl.ds(c1, C)], idx_b)
      gb = pltpu.async_copy(table.at[idx_b], rows_b, sem_b)
      pltpu.make_async_copy(table.at[idx_a], rows_a, sem_a).wait()
      pltpu.sync_copy(rows_a, seq_out.at[pl.ds(c0, C)])
      pltpu.sync_copy(sidx.at[pl.ds(c2, C)], idx_a)
      pltpu.async_copy(table.at[idx_a], rows_a, sem_a)
      gb.wait()
      pltpu.sync_copy(rows_b, seq_out.at[pl.ds(c1, C)])
      return carry

    lax.fori_loop(0, n_pairs, body, 0)
    pltpu.make_async_copy(table.at[idx_a], rows_a, sem_a).wait()

  return gather_k


def _make_sc_transpose(V, D):
  info = plsc.get_sparse_core_info()
  NC, NS = info.num_cores, info.num_subcores
  NW = NC * NS
  CB = 800
  KB = CB // 16
  nch = V // CB
  assert nch * CB == V
  nloop = (nch + NW - 1) // NW

  mesh = plsc.VectorSubcoreMesh(core_axis_name="c", subcore_axis_name="s")

  @functools.partial(
      pl.kernel, mesh=mesh,
      out_type=jax.ShapeDtypeStruct((V, D), jnp.float32),
      scratch_types=[
          pltpu.VMEM((D, CB), jnp.float32),
          pltpu.VMEM((CB, D), jnp.float32),
      ],
      compiler_params=pltpu.CompilerParams(
          use_tc_tiling_on_sc=False, needs_layout_passes=False),
  )
  def tr_k(table_t, out, buf_in, buf_out):
    wid = lax.axis_index("s") * NC + lax.axis_index("c")
    rows = [lax.iota(jnp.int32, 16) + 16 * kk for kk in range(KB)]

    def body(k, carry):
      c = wid + k * NW

      @pl.when(c < nch)
      def _():
        c0 = c * CB
        pltpu.sync_copy(table_t.at[:, pl.ds(c0, CB)], buf_in)

        def inner(d, carry2):
          col = jnp.full((16,), d, jnp.int32)
          for kk in range(KB):
            v = buf_in[d, pl.ds(kk * 16, 16)]
            plsc.store_scatter(buf_out, [rows[kk], col], v)
          return carry2

        lax.fori_loop(0, D, inner, 0)
        pltpu.sync_copy(buf_out, out.at[pl.ds(c0, CB)])

      return carry

    lax.fori_loop(0, nloop, body, 0)

  return tr_k


def _make_tc_att_mlp(B, L, D, H1, H2, Bb):
  grid = (B // Bb,)
  P = 4
  LP = L // P
  DP = P * D

  def body(uid_ref, tid_ref, padf_ref, attb_ref, eu_ref, et_ref, seq_ref,
           W1_ref, b1_ref, W2_ref, b2_ref, W3_ref, b3_ref,
           BAs_ref, BAm_ref, cAt_ref, ab1_ref, BA2_ref, ab2_ref,
           E_ref, F_ref, out_ref):
    M = Bb * LP
    S = seq_ref[...]
    padf = padf_ref[...]
    mexp = jnp.dot(padf.reshape(M, P), E_ref[...],
                   preferred_element_type=jnp.float32)
    S = S * mexp
    eu = jnp.where(uid_ref[...] != 0, eu_ref[...], 0.0)
    et = jnp.where(tid_ref[...] != 0, et_ref[...], 0.0)
    et4 = jnp.concatenate([et, et, et, et], axis=1)
    T = jnp.broadcast_to(et4[:, None, :], (Bb, LP, DP)).reshape(M, DP)
    c = jnp.dot(et, cAt_ref[...], preferred_element_type=jnp.float32)
    c = c + ab1_ref[...][None, :]
    C2 = jnp.broadcast_to(c[:, None, :], (Bb, LP, DP)).reshape(M, DP)
    h = jnp.dot(S, BAs_ref[...], preferred_element_type=jnp.float32)
    h = h + jnp.dot(S * T, BAm_ref[...], preferred_element_type=jnp.float32)
    h = jnp.maximum(h + C2, 0.0)
    lg = jnp.dot(h, BA2_ref[...], preferred_element_type=jnp.float32)
    lg = (lg + ab2_ref[...][None, :]).reshape(Bb, LP, P)
    e = jnp.exp(lg + attb_ref[...])
    s = jnp.sum(jnp.sum(e, axis=2, keepdims=True), axis=1, keepdims=True)
    we = jnp.dot(e.reshape(M, P), E_ref[...],
                 preferred_element_type=jnp.float32)
    ap = jnp.sum((we * S).reshape(Bb, LP, DP), axis=1)
    att = jnp.dot(ap, F_ref[...], preferred_element_type=jnp.float32)
    att = att / s[:, :, 0]
    x = jnp.concatenate([eu, et, att], axis=1)
    x = jnp.dot(x, W1_ref[...], preferred_element_type=jnp.float32)
    x = jnp.maximum(x + b1_ref[...][None, :], 0.0)
    x = jnp.dot(x, W2_ref[...], preferred_element_type=jnp.float32)
    x = jnp.maximum(x + b2_ref[...][None, :], 0.0)
    y = jnp.dot(x, W3_ref[...], preferred_element_type=jnp.float32)
    out_ref[...] = y + b3_ref[...][None, :]

  full = lambda *shape: pl.BlockSpec(shape, lambda i: (0,) * len(shape))
  in_specs = [
      pl.BlockSpec((Bb, 1), lambda i: (i, 0)),
      pl.BlockSpec((Bb, 1), lambda i: (i, 0)),
      pl.BlockSpec((Bb, LP, P), lambda i: (i, 0, 0)),
      pl.BlockSpec((Bb, LP, P), lambda i: (i, 0, 0)),
      pl.BlockSpec((Bb, D), lambda i: (i, 0)),
      pl.BlockSpec((Bb, D), lambda i: (i, 0)),
      pl.BlockSpec((Bb * LP, DP), lambda i: (i, 0)),
      full(3 * D, H1), full(H1), full(H1, H2), full(H2),
      full(H2, 1), full(1),
      full(DP, DP), full(DP, DP), full(D, DP), full(DP),
      full(DP, P), full(P),
      full(P, DP), full(DP, D),
  ]

  return pl.pallas_call(
      body,
      grid=grid,
      in_specs=in_specs,
      out_specs=pl.BlockSpec((Bb, 1), lambda i: (i, 0)),
      out_shape=jax.ShapeDtypeStruct((B, 1), jnp.float32),
      compiler_params=pltpu.CompilerParams(
          dimension_semantics=("arbitrary",),
      ),
  )


def _block_diag4(A):
  K, N = A.shape
  out = jnp.zeros((4 * K, 4 * N), A.dtype)
  for i in range(4):
    out = out.at[i * K:(i + 1) * K, i * N:(i + 1) * N].set(A)
  return out


def kernel(user_id, target_brand_id, pay_brand_seq, pay_brand_seq_mask,
           emb, W1, b1, W2, b2, W3, b3, aW1, ab1, aW2, ab2):
  B, L = pay_brand_seq.shape
  V, D = emb.shape
  H1 = W1.shape[1]
  H2 = W2.shape[1]
  P = 4
  LP = L // P

  uid = user_id.reshape(B)
  tid = target_brand_id.reshape(B)
  sidx = pay_brand_seq.reshape(B * L)

  table_lin = _make_sc_transpose(V, D)(emb.T)

  A1, A2, A3, A4 = aW1[:D], aW1[D:2 * D], aW1[2 * D:3 * D], aW1[3 * D:]
  BAs = _block_diag4(A1 + A3)
  BAm = _block_diag4(A4)
  cAt = A2 - A3
  ab1t = jnp.tile(ab1, P)
  BA2 = _block_diag4(aW2)
  ab2t = jnp.tile(ab2, P)
  eye = jnp.eye(D, dtype=jnp.float32)
  E = jnp.kron(jnp.eye(P, dtype=jnp.float32), jnp.ones((1, D), jnp.float32))
  F = jnp.concatenate([eye] * P, axis=0)

  padf = (pay_brand_seq != 0).astype(jnp.float32).reshape(B, LP, P)
  attb = jnp.where(pay_brand_seq_mask == 0, -jnp.inf, 0.0
                   ).astype(jnp.float32).reshape(B, LP, P)

  NS = 2
  Bc = B // NS
  gather = _make_sc_gather(V, D, Bc, L)
  tc = _make_tc_att_mlp(Bc, L, D, H1, H2, Bb=64)
  outs = []
  for ci in range(NS):
    b0 = ci * Bc
    e_user, e_tgt, seq_flat = gather(
        table_lin,
        lax.slice_in_dim(uid, b0, b0 + Bc),
        lax.slice_in_dim(tid, b0, b0 + Bc),
        lax.slice_in_dim(sidx, b0 * L, (b0 + Bc) * L))
    seq_p = seq_flat.reshape(Bc * LP, P * D)
    outs.append(tc(
        lax.slice_in_dim(user_id, b0, b0 + Bc),
        lax.slice_in_dim(target_brand_id, b0, b0 + Bc),
        lax.slice_in_dim(padf, b0, b0 + Bc),
        lax.slice_in_dim(attb, b0, b0 + Bc),
        e_user, e_tgt, seq_p,
        W1, b1, W2, b2, W3, b3,
        BAs, BAm, cAt, ab1t, BA2, ab2t, E, F))
  return jnp.concatenate(outs, axis=0)


# speedup vs baseline: 2.0819x; 2.0819x over previous
import functools

import jax
import jax.numpy as jnp
import numpy as np
from jax import lax
from jax.experimental import pallas as pl
from jax.experimental.pallas import tpu as pltpu
from jax.experimental.pallas import tpu_sc as plsc


def _make_sc_gather(V, D, B, L):
  info = plsc.get_sparse_core_info()
  NC, NS = info.num_cores, info.num_subcores
  NW = NC * NS
  n_seq = B * L
  assert n_seq % NW == 0 and B % NW == 0
  seq_per_w = n_seq // NW
  C = 1024
  assert seq_per_w % (2 * C) == 0
  n_pairs = seq_per_w // (2 * C)
  b_per_w = B // NW

  mesh = plsc.VectorSubcoreMesh(core_axis_name="c", subcore_axis_name="s")

  @functools.partial(
      pl.kernel, mesh=mesh,
      out_type=(
          jax.ShapeDtypeStruct((B, D), jnp.float32),
          jax.ShapeDtypeStruct((B, D), jnp.float32),
          jax.ShapeDtypeStruct((n_seq, D), jnp.float32),
      ),
      scratch_types=[
          pltpu.VMEM((C,), jnp.int32),
          pltpu.VMEM((C,), jnp.int32),
          pltpu.VMEM((C, D), jnp.float32),
          pltpu.VMEM((C, D), jnp.float32),
          pltpu.VMEM((b_per_w,), jnp.int32),
          pltpu.VMEM((b_per_w, D), jnp.float32),
          pltpu.SemaphoreType.DMA,
          pltpu.SemaphoreType.DMA,
      ],
      compiler_params=pltpu.CompilerParams(use_tc_tiling_on_sc=False),
  )
  def gather_k(table, uid, tid, sidx, e_user, e_tgt, seq_out,
               idx_a, idx_b, rows_a, rows_b, sid_v, srow_v, sem_a, sem_b):
    wid = lax.axis_index("s") * NC + lax.axis_index("c")
    ub = wid * b_per_w
    pltpu.sync_copy(uid.at[pl.ds(ub, b_per_w)], sid_v)
    pltpu.async_copy(table.at[sid_v], srow_v, sem_a).wait()
    pltpu.sync_copy(srow_v, e_user.at[pl.ds(ub, b_per_w)])
    pltpu.sync_copy(tid.at[pl.ds(ub, b_per_w)], sid_v)
    pltpu.async_copy(table.at[sid_v], srow_v, sem_a).wait()
    pltpu.sync_copy(srow_v, e_tgt.at[pl.ds(ub, b_per_w)])

    sb = wid * seq_per_w
    last = sb + seq_per_w - C

    pltpu.sync_copy(sidx.at[pl.ds(sb, C)], idx_a)
    ga0 = pltpu.async_copy(table.at[idx_a], rows_a, sem_a)

    def body(j, carry):
      c0 = sb + (2 * j) * C
      c1 = c0 + C
      c2 = jnp.minimum(c0 + 2 * C, last)
      pltpu.sync_copy(sidx.at[pl.ds(c1, C)], idx_b)
      gb = pltpu.async_copy(table.at[idx_b], rows_b, sem_b)
      pltpu.make_async_copy(table.at[idx_a], rows_a, sem_a).wait()
      pltpu.sync_copy(rows_a, seq_out.at[pl.ds(c0, C)])
      pltpu.sync_copy(sidx.at[pl.ds(c2, C)], idx_a)
      pltpu.async_copy(table.at[idx_a], rows_a, sem_a)
      gb.wait()
      pltpu.sync_copy(rows_b, seq_out.at[pl.ds(c1, C)])
      return carry

    lax.fori_loop(0, n_pairs, body, 0)
    pltpu.make_async_copy(table.at[idx_a], rows_a, sem_a).wait()

  return gather_k


def _make_sc_transpose(V, D):
  info = plsc.get_sparse_core_info()
  NC, NS = info.num_cores, info.num_subcores
  NW = NC * NS
  CB = 800
  KB = CB // 16
  nch = V // CB
  assert nch * CB == V
  nloop = (nch + NW - 1) // NW

  mesh = plsc.VectorSubcoreMesh(core_axis_name="c", subcore_axis_name="s")

  @functools.partial(
      pl.kernel, mesh=mesh,
      out_type=jax.ShapeDtypeStruct((V, D), jnp.float32),
      scratch_types=[
          pltpu.VMEM((D, CB), jnp.float32),
          pltpu.VMEM((CB, D), jnp.float32),
      ],
      compiler_params=pltpu.CompilerParams(
          use_tc_tiling_on_sc=False, needs_layout_passes=False),
  )
  def tr_k(table_t, out, buf_in, buf_out):
    wid = lax.axis_index("s") * NC + lax.axis_index("c")
    rows = [lax.iota(jnp.int32, 16) + 16 * kk for kk in range(KB)]

    def body(k, carry):
      c = wid + k * NW

      @pl.when(c < nch)
      def _():
        c0 = c * CB
        pltpu.sync_copy(table_t.at[:, pl.ds(c0, CB)], buf_in)

        def inner(d, carry2):
          col = jnp.full((16,), d, jnp.int32)
          for kk in range(KB):
            v = buf_in[d, pl.ds(kk * 16, 16)]
            plsc.store_scatter(buf_out, [rows[kk], col], v)
          return carry2

        lax.fori_loop(0, D, inner, 0)
        pltpu.sync_copy(buf_out, out.at[pl.ds(c0, CB)])

      return carry

    lax.fori_loop(0, nloop, body, 0)

  return tr_k


def _make_tc_att_mlp(B, L, D, H1, H2, Bb):
  grid = (B // Bb,)
  P = 4
  LP = L // P
  DP = P * D

  def body(uid_ref, tid_ref, padf_ref, attb_ref, eu_ref, et_ref, seq_ref,
           W1_ref, b1_ref, W2_ref, b2_ref, W3_ref, b3_ref,
           BAs_ref, BAm_ref, cAt_ref, ab1_ref, BA2_ref, ab2_ref,
           E_ref, F_ref, out_ref):
    M = Bb * LP
    S = seq_ref[...]
    padf = padf_ref[...]
    mexp = jnp.dot(padf.reshape(M, P), E_ref[...],
                   preferred_element_type=jnp.float32)
    S = S * mexp
    eu = jnp.where(uid_ref[...] != 0, eu_ref[...], 0.0)
    et = jnp.where(tid_ref[...] != 0, et_ref[...], 0.0)
    et4 = jnp.concatenate([et, et, et, et], axis=1)
    T = jnp.broadcast_to(et4[:, None, :], (Bb, LP, DP)).reshape(M, DP)
    c = jnp.dot(et, cAt_ref[...], preferred_element_type=jnp.float32)
    c = c + ab1_ref[...][None, :]
    C2 = jnp.broadcast_to(c[:, None, :], (Bb, LP, DP)).reshape(M, DP)
    h = jnp.dot(S, BAs_ref[...], preferred_element_type=jnp.float32)
    h = h + jnp.dot(S * T, BAm_ref[...], preferred_element_type=jnp.float32)
    h = jnp.maximum(h + C2, 0.0)
    lg = jnp.dot(h, BA2_ref[...], preferred_element_type=jnp.float32)
    lg = (lg + ab2_ref[...][None, :]).reshape(Bb, LP, P)
    e = jnp.exp(lg + attb_ref[...])
    s = jnp.sum(jnp.sum(e, axis=2, keepdims=True), axis=1, keepdims=True)
    we = jnp.dot(e.reshape(M, P), E_ref[...],
                 preferred_element_type=jnp.float32)
    ap = jnp.sum((we * S).reshape(Bb, LP, DP), axis=1)
    att = jnp.dot(ap, F_ref[...], preferred_element_type=jnp.float32)
    att = att / s[:, :, 0]
    x = jnp.concatenate([eu, et, att], axis=1)
    x = jnp.dot(x, W1_ref[...], preferred_element_type=jnp.float32)
    x = jnp.maximum(x + b1_ref[...][None, :], 0.0)
    x = jnp.dot(x, W2_ref[...], preferred_element_type=jnp.float32)
    x = jnp.maximum(x + b2_ref[...][None, :], 0.0)
    y = jnp.dot(x, W3_ref[...], preferred_element_type=jnp.float32)
    out_ref[...] = y + b3_ref[...][None, :]

  full = lambda *shape: pl.BlockSpec(shape, lambda i: (0,) * len(shape))
  in_specs = [
      pl.BlockSpec((Bb, 1), lambda i: (i, 0)),
      pl.BlockSpec((Bb, 1), lambda i: (i, 0)),
      pl.BlockSpec((Bb, LP, P), lambda i: (i, 0, 0)),
      pl.BlockSpec((Bb, LP, P), lambda i: (i, 0, 0)),
      pl.BlockSpec((Bb, D), lambda i: (i, 0)),
      pl.BlockSpec((Bb, D), lambda i: (i, 0)),
      pl.BlockSpec((Bb * LP, DP), lambda i: (i, 0)),
      full(3 * D, H1), full(H1), full(H1, H2), full(H2),
      full(H2, 1), full(1),
      full(DP, DP), full(DP, DP), full(D, DP), full(DP),
      full(DP, P), full(P),
      full(P, DP), full(DP, D),
  ]

  return pl.pallas_call(
      body,
      grid=grid,
      in_specs=in_specs,
      out_specs=pl.BlockSpec((Bb, 1), lambda i: (i, 0)),
      out_shape=jax.ShapeDtypeStruct((B, 1), jnp.float32),
      compiler_params=pltpu.CompilerParams(
          dimension_semantics=("arbitrary",),
      ),
  )


def _block_diag4(A):
  K, N = A.shape
  out = jnp.zeros((4 * K, 4 * N), A.dtype)
  for i in range(4):
    out = out.at[i * K:(i + 1) * K, i * N:(i + 1) * N].set(A)
  return out


def kernel(user_id, target_brand_id, pay_brand_seq, pay_brand_seq_mask,
           emb, W1, b1, W2, b2, W3, b3, aW1, ab1, aW2, ab2):
  B, L = pay_brand_seq.shape
  V, D = emb.shape
  H1 = W1.shape[1]
  H2 = W2.shape[1]
  P = 4
  LP = L // P

  uid = user_id.reshape(B)
  tid = target_brand_id.reshape(B)
  sidx = pay_brand_seq.reshape(B * L)

  emb_p = lax.optimization_barrier(emb.reshape(V * D // 128, 128))
  table_lin = emb_p.reshape(V, D)

  A1, A2, A3, A4 = aW1[:D], aW1[D:2 * D], aW1[2 * D:3 * D], aW1[3 * D:]
  BAs = _block_diag4(A1 + A3)
  BAm = _block_diag4(A4)
  cAt = A2 - A3
  ab1t = jnp.tile(ab1, P)
  BA2 = _block_diag4(aW2)
  ab2t = jnp.tile(ab2, P)
  eye = jnp.eye(D, dtype=jnp.float32)
  E = jnp.kron(jnp.eye(P, dtype=jnp.float32), jnp.ones((1, D), jnp.float32))
  F = jnp.concatenate([eye] * P, axis=0)

  padf = (pay_brand_seq != 0).astype(jnp.float32).reshape(B, LP, P)
  attb = jnp.where(pay_brand_seq_mask == 0, -jnp.inf, 0.0
                   ).astype(jnp.float32).reshape(B, LP, P)

  NS = 2
  Bc = B // NS
  gather = _make_sc_gather(V, D, Bc, L)
  tc = _make_tc_att_mlp(Bc, L, D, H1, H2, Bb=64)
  outs = []
  for ci in range(NS):
    b0 = ci * Bc
    e_user, e_tgt, seq_flat = gather(
        table_lin,
        lax.slice_in_dim(uid, b0, b0 + Bc),
        lax.slice_in_dim(tid, b0, b0 + Bc),
        lax.slice_in_dim(sidx, b0 * L, (b0 + Bc) * L))
    seq_p = seq_flat.reshape(Bc * LP, P * D)
    outs.append(tc(
        lax.slice_in_dim(user_id, b0, b0 + Bc),
        lax.slice_in_dim(target_brand_id, b0, b0 + Bc),
        lax.slice_in_dim(padf, b0, b0 + Bc),
        lax.slice_in_dim(attb, b0, b0 + Bc),
        e_user, e_tgt, seq_p,
        W1, b1, W2, b2, W3, b3,
        BAs, BAm, cAt, ab1t, BA2, ab2t, E, F))
  return jnp.concatenate(outs, axis=0)


# speedup vs baseline: 2.1519x; 1.0336x over previous
import functools

import jax
import jax.numpy as jnp
import numpy as np
from jax import lax
from jax.experimental import pallas as pl
from jax.experimental.pallas import tpu as pltpu
from jax.experimental.pallas import tpu_sc as plsc


def _make_sc_gather(V, D, B, L):
  info = plsc.get_sparse_core_info()
  NC, NS = info.num_cores, info.num_subcores
  NW = NC * NS
  n_seq = B * L
  assert n_seq % NW == 0 and B % NW == 0
  seq_per_w = n_seq // NW
  C = 1024
  assert seq_per_w % (2 * C) == 0
  n_pairs = seq_per_w // (2 * C)
  b_per_w = B // NW

  mesh = plsc.VectorSubcoreMesh(core_axis_name="c", subcore_axis_name="s")

  @functools.partial(
      pl.kernel, mesh=mesh,
      out_type=(
          jax.ShapeDtypeStruct((B, D), jnp.float32),
          jax.ShapeDtypeStruct((B, D), jnp.float32),
          jax.ShapeDtypeStruct((n_seq, D), jnp.float32),
      ),
      scratch_types=[
          pltpu.VMEM((C,), jnp.int32),
          pltpu.VMEM((C,), jnp.int32),
          pltpu.VMEM((C, D), jnp.float32),
          pltpu.VMEM((C, D), jnp.float32),
          pltpu.VMEM((b_per_w,), jnp.int32),
          pltpu.VMEM((b_per_w, D), jnp.float32),
          pltpu.SemaphoreType.DMA,
          pltpu.SemaphoreType.DMA,
      ],
      compiler_params=pltpu.CompilerParams(use_tc_tiling_on_sc=False),
  )
  def gather_k(table, uid, tid, sidx, e_user, e_tgt, seq_out,
               idx_a, idx_b, rows_a, rows_b, sid_v, srow_v, sem_a, sem_b):
    wid = lax.axis_index("s") * NC + lax.axis_index("c")
    ub = wid * b_per_w
    pltpu.sync_copy(uid.at[pl.ds(ub, b_per_w)], sid_v)
    pltpu.async_copy(table.at[sid_v], srow_v, sem_a).wait()
    pltpu.sync_copy(srow_v, e_user.at[pl.ds(ub, b_per_w)])
    pltpu.sync_copy(tid.at[pl.ds(ub, b_per_w)], sid_v)
    pltpu.async_copy(table.at[sid_v], srow_v, sem_a).wait()
    pltpu.sync_copy(srow_v, e_tgt.at[pl.ds(ub, b_per_w)])

    sb = wid * seq_per_w
    last = sb + seq_per_w - C

    pltpu.sync_copy(sidx.at[pl.ds(sb, C)], idx_a)
    ga0 = pltpu.async_copy(table.at[idx_a], rows_a, sem_a)

    def body(j, carry):
      c0 = sb + (2 * j) * C
      c1 = c0 + C
      c2 = jnp.minimum(c0 + 2 * C, last)
      pltpu.sync_copy(sidx.at[pl.ds(c1, C)], idx_b)
      gb = pltpu.async_copy(table.at[idx_b], rows_b, sem_b)
      pltpu.make_async_copy(table.at[idx_a], rows_a, sem_a).wait()
      pltpu.sync_copy(rows_a, seq_out.at[pl.ds(c0, C)])
      pltpu.sync_copy(sidx.at[pl.ds(c2, C)], idx_a)
      pltpu.async_copy(table.at[idx_a], rows_a, sem_a)
      gb.wait()
      pltpu.sync_copy(rows_b, seq_out.at[pl.ds(c1, C)])
      return carry

    lax.fori_loop(0, n_pairs, body, 0)
    pltpu.make_async_copy(table.at[idx_a], rows_a, sem_a).wait()

  return gather_k


def _make_sc_transpose(V, D):
  info = plsc.get_sparse_core_info()
  NC, NS = info.num_cores, info.num_subcores
  NW = NC * NS
  CB = 800
  KB = CB // 16
  nch = V // CB
  assert nch * CB == V
  nloop = (nch + NW - 1) // NW

  mesh = plsc.VectorSubcoreMesh(core_axis_name="c", subcore_axis_name="s")

  @functools.partial(
      pl.kernel, mesh=mesh,
      out_type=jax.ShapeDtypeStruct((V, D), jnp.float32),
      scratch_types=[
          pltpu.VMEM((D, CB), jnp.float32),
          pltpu.VMEM((CB, D), jnp.float32),
      ],
      compiler_params=pltpu.CompilerParams(
          use_tc_tiling_on_sc=False, needs_layout_passes=False),
  )
  def tr_k(table_t, out, buf_in, buf_out):
    wid = lax.axis_index("s") * NC + lax.axis_index("c")
    rows = [lax.iota(jnp.int32, 16) + 16 * kk for kk in range(KB)]

    def body(k, carry):
      c = wid + k * NW

      @pl.when(c < nch)
      def _():
        c0 = c * CB
        pltpu.sync_copy(table_t.at[:, pl.ds(c0, CB)], buf_in)

        def inner(d, carry2):
          col = jnp.full((16,), d, jnp.int32)
          for kk in range(KB):
            v = buf_in[d, pl.ds(kk * 16, 16)]
            plsc.store_scatter(buf_out, [rows[kk], col], v)
          return carry2

        lax.fori_loop(0, D, inner, 0)
        pltpu.sync_copy(buf_out, out.at[pl.ds(c0, CB)])

      return carry

    lax.fori_loop(0, nloop, body, 0)

  return tr_k


def _make_tc_att_mlp(B, L, D, H1, H2, Bb, P):
  grid = (B // Bb,)
  LP = L // P
  DP = P * D

  def body(uid_ref, tid_ref, padf_ref, attb_ref, eu_ref, et_ref, seq_ref,
           W1_ref, b1_ref, W2_ref, b2_ref, W3_ref, b3_ref,
           BAs_ref, BAm_ref, cAt_ref, ab1_ref, BA2_ref, ab2_ref,
           E_ref, F_ref, out_ref):
    M = Bb * LP
    S = seq_ref[...]
    padf = padf_ref[...]
    mexp = jnp.dot(padf.reshape(M, P), E_ref[...],
                   preferred_element_type=jnp.float32)
    S = S * mexp
    eu = jnp.where(uid_ref[...] != 0, eu_ref[...], 0.0)
    et = jnp.where(tid_ref[...] != 0, et_ref[...], 0.0)
    et4 = jnp.concatenate([et] * P, axis=1)
    T = jnp.broadcast_to(et4[:, None, :], (Bb, LP, DP)).reshape(M, DP)
    c = jnp.dot(et, cAt_ref[...], preferred_element_type=jnp.float32)
    c = c + ab1_ref[...][None, :]
    C2 = jnp.broadcast_to(c[:, None, :], (Bb, LP, DP)).reshape(M, DP)
    h = jnp.dot(S, BAs_ref[...], preferred_element_type=jnp.float32)
    h = h + jnp.dot(S * T, BAm_ref[...], preferred_element_type=jnp.float32)
    h = jnp.maximum(h + C2, 0.0)
    lg = jnp.dot(h, BA2_ref[...], preferred_element_type=jnp.float32)
    lg = (lg + ab2_ref[...][None, :]).reshape(Bb, LP, P)
    e = jnp.exp(lg + attb_ref[...])
    s = jnp.sum(jnp.sum(e, axis=2, keepdims=True), axis=1, keepdims=True)
    we = jnp.dot(e.reshape(M, P), E_ref[...],
                 preferred_element_type=jnp.float32)
    ap = jnp.sum((we * S).reshape(Bb, LP, DP), axis=1)
    att = jnp.dot(ap, F_ref[...], preferred_element_type=jnp.float32)
    att = att / s[:, :, 0]
    x = jnp.concatenate([eu, et, att], axis=1)
    x = jnp.dot(x, W1_ref[...], preferred_element_type=jnp.float32)
    x = jnp.maximum(x + b1_ref[...][None, :], 0.0)
    x = jnp.dot(x, W2_ref[...], preferred_element_type=jnp.float32)
    x = jnp.maximum(x + b2_ref[...][None, :], 0.0)
    y = jnp.dot(x, W3_ref[...], preferred_element_type=jnp.float32)
    out_ref[...] = y + b3_ref[...][None, :]

  full = lambda *shape: pl.BlockSpec(shape, lambda i: (0,) * len(shape))
  in_specs = [
      pl.BlockSpec((Bb, 1), lambda i: (i, 0)),
      pl.BlockSpec((Bb, 1), lambda i: (i, 0)),
      pl.BlockSpec((Bb, LP, P), lambda i: (i, 0, 0)),
      pl.BlockSpec((Bb, LP, P), lambda i: (i, 0, 0)),
      pl.BlockSpec((Bb, D), lambda i: (i, 0)),
      pl.BlockSpec((Bb, D), lambda i: (i, 0)),
      pl.BlockSpec((Bb * LP, DP), lambda i: (i, 0)),
      full(3 * D, H1), full(H1), full(H1, H2), full(H2),
      full(H2, 1), full(1),
      full(DP, DP), full(DP, DP), full(D, DP), full(DP),
      full(DP, P), full(P),
      full(P, DP), full(DP, D),
  ]

  return pl.pallas_call(
      body,
      grid=grid,
      in_specs=in_specs,
      out_specs=pl.BlockSpec((Bb, 1), lambda i: (i, 0)),
      out_shape=jax.ShapeDtypeStruct((B, 1), jnp.float32),
      compiler_params=pltpu.CompilerParams(
          dimension_semantics=("arbitrary",),
      ),
  )


def _block_diagp(A, P):
  K, N = A.shape
  out = jnp.zeros((P * K, P * N), A.dtype)
  for i in range(P):
    out = out.at[i * K:(i + 1) * K, i * N:(i + 1) * N].set(A)
  return out


def kernel(user_id, target_brand_id, pay_brand_seq, pay_brand_seq_mask,
           emb, W1, b1, W2, b2, W3, b3, aW1, ab1, aW2, ab2):
  B, L = pay_brand_seq.shape
  V, D = emb.shape
  H1 = W1.shape[1]
  H2 = W2.shape[1]
  P = 4
  LP = L // P

  uid = user_id.reshape(B)
  tid = target_brand_id.reshape(B)
  sidx = pay_brand_seq.reshape(B * L)

  emb_p = lax.optimization_barrier(emb.reshape(V * D // 128, 128))
  table_lin = emb_p.reshape(V, D)

  A1, A2, A3, A4 = aW1[:D], aW1[D:2 * D], aW1[2 * D:3 * D], aW1[3 * D:]
  BAs = _block_diagp(A1 + A3, P)
  BAm = _block_diagp(A4, P)
  cAt = A2 - A3
  ab1t = jnp.tile(ab1, P)
  BA2 = _block_diagp(aW2, P)
  ab2t = jnp.tile(ab2, P)
  eye = jnp.eye(D, dtype=jnp.float32)
  E = jnp.kron(jnp.eye(P, dtype=jnp.float32), jnp.ones((1, D), jnp.float32))
  F = jnp.concatenate([eye] * P, axis=0)

  padf = (pay_brand_seq != 0).astype(jnp.float32).reshape(B, LP, P)
  attb = jnp.where(pay_brand_seq_mask == 0, -jnp.inf, 0.0
                   ).astype(jnp.float32).reshape(B, LP, P)

  gather = _make_sc_gather(V, D, B, L)
  e_user, e_tgt, seq_flat = gather(table_lin, uid, tid, sidx)
  seq_p = seq_flat.reshape(B * LP, P * D)

  tc = _make_tc_att_mlp(B, L, D, H1, H2, Bb=64, P=P)
  return tc(user_id, target_brand_id, padf, attb,
            e_user, e_tgt, seq_p,
            W1, b1, W2, b2, W3, b3,
            BAs, BAm, cAt, ab1t, BA2, ab2t, E, F)


# speedup vs baseline: 2.2488x; 1.0450x over previous
import functools

import jax
import jax.numpy as jnp
import numpy as np
from jax import lax
from jax.experimental import pallas as pl
from jax.experimental.pallas import tpu as pltpu
from jax.experimental.pallas import tpu_sc as plsc


def _make_sc_gather(V, D, B, L):
  info = plsc.get_sparse_core_info()
  NC, NS = info.num_cores, info.num_subcores
  NW = NC * NS
  n_seq = B * L
  assert n_seq % NW == 0 and B % NW == 0
  seq_per_w = n_seq // NW
  C = 1280
  assert seq_per_w % (2 * C) == 0
  n_pairs = seq_per_w // (2 * C)
  b_per_w = B // NW

  mesh = plsc.VectorSubcoreMesh(core_axis_name="c", subcore_axis_name="s")

  @functools.partial(
      pl.kernel, mesh=mesh,
      out_type=(
          jax.ShapeDtypeStruct((B, D), jnp.float32),
          jax.ShapeDtypeStruct((B, D), jnp.float32),
          jax.ShapeDtypeStruct((n_seq, D), jnp.float32),
      ),
      scratch_types=[
          pltpu.VMEM((C,), jnp.int32),
          pltpu.VMEM((C,), jnp.int32),
          pltpu.VMEM((C, D), jnp.float32),
          pltpu.VMEM((C, D), jnp.float32),
          pltpu.VMEM((b_per_w,), jnp.int32),
          pltpu.VMEM((b_per_w, D), jnp.float32),
          pltpu.SemaphoreType.DMA,
          pltpu.SemaphoreType.DMA,
      ],
      compiler_params=pltpu.CompilerParams(use_tc_tiling_on_sc=False),
  )
  def gather_k(table, uid, tid, sidx, e_user, e_tgt, seq_out,
               idx_a, idx_b, rows_a, rows_b, sid_v, srow_v, sem_a, sem_b):
    wid = lax.axis_index("s") * NC + lax.axis_index("c")
    ub = wid * b_per_w
    pltpu.sync_copy(uid.at[pl.ds(ub, b_per_w)], sid_v)
    pltpu.async_copy(table.at[sid_v], srow_v, sem_a).wait()
    pltpu.sync_copy(srow_v, e_user.at[pl.ds(ub, b_per_w)])
    pltpu.sync_copy(tid.at[pl.ds(ub, b_per_w)], sid_v)
    pltpu.async_copy(table.at[sid_v], srow_v, sem_a).wait()
    pltpu.sync_copy(srow_v, e_tgt.at[pl.ds(ub, b_per_w)])

    sb = wid * seq_per_w
    last = sb + seq_per_w - C

    pltpu.sync_copy(sidx.at[pl.ds(sb, C)], idx_a)
    ga0 = pltpu.async_copy(table.at[idx_a], rows_a, sem_a)

    def body(j, carry):
      c0 = sb + (2 * j) * C
      c1 = c0 + C
      c2 = jnp.minimum(c0 + 2 * C, last)
      pltpu.sync_copy(sidx.at[pl.ds(c1, C)], idx_b)
      gb = pltpu.async_copy(table.at[idx_b], rows_b, sem_b)
      pltpu.make_async_copy(table.at[idx_a], rows_a, sem_a).wait()
      pltpu.sync_copy(rows_a, seq_out.at[pl.ds(c0, C)])
      pltpu.sync_copy(sidx.at[pl.ds(c2, C)], idx_a)
      pltpu.async_copy(table.at[idx_a], rows_a, sem_a)
      gb.wait()
      pltpu.sync_copy(rows_b, seq_out.at[pl.ds(c1, C)])
      return carry

    lax.fori_loop(0, n_pairs, body, 0)
    pltpu.make_async_copy(table.at[idx_a], rows_a, sem_a).wait()

  return gather_k


def _make_sc_transpose(V, D):
  info = plsc.get_sparse_core_info()
  NC, NS = info.num_cores, info.num_subcores
  NW = NC * NS
  CB = 800
  KB = CB // 16
  nch = V // CB
  assert nch * CB == V
  nloop = (nch + NW - 1) // NW

  mesh = plsc.VectorSubcoreMesh(core_axis_name="c", subcore_axis_name="s")

  @functools.partial(
      pl.kernel, mesh=mesh,
      out_type=jax.ShapeDtypeStruct((V, D), jnp.float32),
      scratch_types=[
          pltpu.VMEM((D, CB), jnp.float32),
          pltpu.VMEM((CB, D), jnp.float32),
      ],
      compiler_params=pltpu.CompilerParams(
          use_tc_tiling_on_sc=False, needs_layout_passes=False),
  )
  def tr_k(table_t, out, buf_in, buf_out):
    wid = lax.axis_index("s") * NC + lax.axis_index("c")
    rows = [lax.iota(jnp.int32, 16) + 16 * kk for kk in range(KB)]

    def body(k, carry):
      c = wid + k * NW

      @pl.when(c < nch)
      def _():
        c0 = c * CB
        pltpu.sync_copy(table_t.at[:, pl.ds(c0, CB)], buf_in)

        def inner(d, carry2):
          col = jnp.full((16,), d, jnp.int32)
          for kk in range(KB):
            v = buf_in[d, pl.ds(kk * 16, 16)]
            plsc.store_scatter(buf_out, [rows[kk], col], v)
          return carry2

        lax.fori_loop(0, D, inner, 0)
        pltpu.sync_copy(buf_out, out.at[pl.ds(c0, CB)])

      return carry

    lax.fori_loop(0, nloop, body, 0)

  return tr_k


def _make_tc_att_mlp(B, L, D, H1, H2, Bb, P):
  grid = (B // Bb,)
  LP = L // P
  DP = P * D

  def body(uid_ref, tid_ref, padf_ref, attb_ref, eu_ref, et_ref, seq_ref,
           W1_ref, b1_ref, W2_ref, b2_ref, W3_ref, b3_ref,
           BAs_ref, BAm_ref, cAt_ref, ab1_ref, BA2_ref, ab2_ref,
           E_ref, F_ref, out_ref):
    M = Bb * LP
    S = seq_ref[...]
    padf = padf_ref[...]
    mexp = jnp.dot(padf.reshape(M, P), E_ref[...],
                   preferred_element_type=jnp.float32)
    S = S * mexp
    eu = jnp.where(uid_ref[...] != 0, eu_ref[...], 0.0)
    et = jnp.where(tid_ref[...] != 0, et_ref[...], 0.0)
    et4 = jnp.concatenate([et] * P, axis=1)
    T = jnp.broadcast_to(et4[:, None, :], (Bb, LP, DP)).reshape(M, DP)
    c = jnp.dot(et, cAt_ref[...], preferred_element_type=jnp.float32)
    c = c + ab1_ref[...][None, :]
    C2 = jnp.broadcast_to(c[:, None, :], (Bb, LP, DP)).reshape(M, DP)
    h = jnp.dot(S, BAs_ref[...], preferred_element_type=jnp.float32)
    h = h + jnp.dot(S * T, BAm_ref[...], preferred_element_type=jnp.float32)
    h = jnp.maximum(h + C2, 0.0)
    lg = jnp.dot(h, BA2_ref[...], preferred_element_type=jnp.float32)
    lg = (lg + ab2_ref[...][None, :]).reshape(Bb, LP, P)
    e = jnp.exp(lg + attb_ref[...])
    s = jnp.sum(jnp.sum(e, axis=2, keepdims=True), axis=1, keepdims=True)
    we = jnp.dot(e.reshape(M, P), E_ref[...],
                 preferred_element_type=jnp.float32)
    ap = jnp.sum((we * S).reshape(Bb, LP, DP), axis=1)
    att = jnp.dot(ap, F_ref[...], preferred_element_type=jnp.float32)
    att = att / s[:, :, 0]
    x = jnp.concatenate([eu, et, att], axis=1)
    x = jnp.dot(x, W1_ref[...], preferred_element_type=jnp.float32)
    x = jnp.maximum(x + b1_ref[...][None, :], 0.0)
    x = jnp.dot(x, W2_ref[...], preferred_element_type=jnp.float32)
    x = jnp.maximum(x + b2_ref[...][None, :], 0.0)
    y = jnp.dot(x, W3_ref[...], preferred_element_type=jnp.float32)
    out_ref[...] = y + b3_ref[...][None, :]

  full = lambda *shape: pl.BlockSpec(shape, lambda i: (0,) * len(shape))
  in_specs = [
      pl.BlockSpec((Bb, 1), lambda i: (i, 0)),
      pl.BlockSpec((Bb, 1), lambda i: (i, 0)),
      pl.BlockSpec((Bb, LP, P), lambda i: (i, 0, 0)),
      pl.BlockSpec((Bb, LP, P), lambda i: (i, 0, 0)),
      pl.BlockSpec((Bb, D), lambda i: (i, 0)),
      pl.BlockSpec((Bb, D), lambda i: (i, 0)),
      pl.BlockSpec((Bb * LP, DP), lambda i: (i, 0)),
      full(3 * D, H1), full(H1), full(H1, H2), full(H2),
      full(H2, 1), full(1),
      full(DP, DP), full(DP, DP), full(D, DP), full(DP),
      full(DP, P), full(P),
      full(P, DP), full(DP, D),
  ]

  return pl.pallas_call(
      body,
      grid=grid,
      in_specs=in_specs,
      out_specs=pl.BlockSpec((Bb, 1), lambda i: (i, 0)),
      out_shape=jax.ShapeDtypeStruct((B, 1), jnp.float32),
      compiler_params=pltpu.CompilerParams(
          dimension_semantics=("arbitrary",),
      ),
  )


def _block_diagp(A, P):
  K, N = A.shape
  out = jnp.zeros((P * K, P * N), A.dtype)
  for i in range(P):
    out = out.at[i * K:(i + 1) * K, i * N:(i + 1) * N].set(A)
  return out


def kernel(user_id, target_brand_id, pay_brand_seq, pay_brand_seq_mask,
           emb, W1, b1, W2, b2, W3, b3, aW1, ab1, aW2, ab2):
  B, L = pay_brand_seq.shape
  V, D = emb.shape
  H1 = W1.shape[1]
  H2 = W2.shape[1]
  P = 4
  LP = L // P

  uid = user_id.reshape(B)
  tid = target_brand_id.reshape(B)
  sidx = pay_brand_seq.reshape(B * L)

  emb_p = lax.optimization_barrier(emb.reshape(V * D // 128, 128))
  table_lin = emb_p.reshape(V, D)

  A1, A2, A3, A4 = aW1[:D], aW1[D:2 * D], aW1[2 * D:3 * D], aW1[3 * D:]
  BAs = _block_diagp(A1 + A3, P)
  BAm = _block_diagp(A4, P)
  cAt = A2 - A3
  ab1t = jnp.tile(ab1, P)
  BA2 = _block_diagp(aW2, P)
  ab2t = jnp.tile(ab2, P)
  eye = jnp.eye(D, dtype=jnp.float32)
  E = jnp.kron(jnp.eye(P, dtype=jnp.float32), jnp.ones((1, D), jnp.float32))
  F = jnp.concatenate([eye] * P, axis=0)

  padf = (pay_brand_seq != 0).astype(jnp.float32).reshape(B, LP, P)
  attb = jnp.where(pay_brand_seq_mask == 0, -jnp.inf, 0.0
                   ).astype(jnp.float32).reshape(B, LP, P)

  gather = _make_sc_gather(V, D, B, L)
  e_user, e_tgt, seq_flat = gather(table_lin, uid, tid, sidx)
  seq_p = seq_flat.reshape(B * LP, P * D)

  tc = _make_tc_att_mlp(B, L, D, H1, H2, Bb=128, P=P)
  return tc(user_id, target_brand_id, padf, attb,
            e_user, e_tgt, seq_p,
            W1, b1, W2, b2, W3, b3,
            BAs, BAm, cAt, ab1t, BA2, ab2t, E, F)
